# baseline (speedup 1.0000x reference)
import functools

import jax
import jax.numpy as jnp
from jax import lax
from jax.experimental import pallas as pl
from jax.experimental.pallas import tpu as pltpu
from jax.experimental.pallas import tpu_sc as plsc

_N = 4096
_C, _H, _W = 128, 7, 7
_NSLAB = _H * _W
_ROWS = _NSLAB * _N
_NC = 2
_NS = 16
_NW = _NC * _NS
_RPW = _N // _NW
_GROUPS = _RPW // 16


def _build():
    mesh = plsc.VectorSubcoreMesh(core_axis_name="c", subcore_axis_name="s")

    @functools.partial(
        pl.kernel,
        mesh=mesh,
        out_type=jax.ShapeDtypeStruct((_NSLAB, _N, _C), jnp.float32),
        scratch_types=[
            pltpu.VMEM((_N,), jnp.int32),
            pltpu.VMEM((1, _RPW), jnp.int32),
            pltpu.VMEM((4, _RPW, _C), jnp.float32),
            pltpu.SemaphoreType.DMA,
            pltpu.SemaphoreType.DMA,
        ],
        compiler_params=pltpu.CompilerParams(needs_layout_passes=False),
    )
    def permute(x_hbm, tgt_hbm, out_hbm, key_v, idx_v, buf_v, sem_in, sem_out):
        wid = lax.axis_index("s") * _NC + lax.axis_index("c")
        base = wid * _RPW

        pltpu.sync_copy(tgt_hbm, key_v)

        for s0 in range(3):
            pltpu.async_copy(
                x_hbm.at[pl.ds(s0 * _N + base, _RPW)], buf_v.at[s0], sem_in)

        iota = lax.iota(jnp.int32, 16)

        def mk(jv, _):
            sl = pl.ds(jv * 16, 16)
            key_v[sl] = key_v[sl] * _N + (jv * 16 + iota)
            return 0
        lax.fori_loop(0, _N // 16, mk, 0)

        ki = [key_v[pl.ds(base + g * 16, 16)] for g in range(_GROUPS)]

        def jbody(jv, accs):
            kv = key_v[pl.ds(jv * 16, 16)]
            accs = list(accs)
            for lane in range(16):
                kj = kv[lane]
                for g in range(_GROUPS):
                    accs[g] = jnp.where(kj < ki[g], accs[g] + 1, accs[g])
            return tuple(accs)

        accs = lax.fori_loop(
            0, _N // 16, jbody,
            tuple(jnp.zeros((16,), jnp.int32) for _ in range(_GROUPS)))

        for g in range(_GROUPS):
            plsc.store_scatter(idx_v, [iota * 0, g * 16 + iota], accs[g])
        ranks = idx_v.at[0]

        def wait_in(b):
            pltpu.make_async_copy(
                x_hbm.at[pl.ds(0, _RPW)], buf_v.at[b], sem_in).wait()

        def wait_out(b):
            pltpu.make_async_copy(
                x_hbm.at[pl.ds(0, _RPW)], buf_v.at[b], sem_out).wait()

        wait_in(0)
        pltpu.async_copy(buf_v.at[0], out_hbm.at[0].at[ranks], sem_out)
        pltpu.async_copy(
            x_hbm.at[pl.ds(3 * _N + base, _RPW)], buf_v.at[3], sem_in)

        def pbody(s, _):
            b = s % 4
            wait_in(b)
            pltpu.async_copy(buf_v.at[b], out_hbm.at[s].at[ranks], sem_out)
            wait_out((s - 1) % 4)
            pltpu.async_copy(
                x_hbm.at[pl.ds((s + 3) * _N + base, _RPW)],
                buf_v.at[(s + 3) % 4], sem_in)
            return 0
        lax.fori_loop(1, _NSLAB - 3, pbody, 0)

        for s in range(_NSLAB - 3, _NSLAB):
            b = s % 4
            wait_in(b)
            pltpu.async_copy(buf_v.at[b], out_hbm.at[s].at[ranks], sem_out)
            wait_out((s - 1) % 4)
        wait_out((_NSLAB - 1) % 4)

    return permute


_permute = _build()


def kernel(rois, target):
    n, c, h, w = rois.shape
    x = rois.transpose(2, 3, 0, 1).reshape(h * w * n, c)
    out = _permute(x, target)
    return out.reshape(h, w, n, c).transpose(2, 3, 0, 1)


# speedup vs baseline: 1.7927x; 1.7927x over previous
import functools

import jax
import jax.numpy as jnp
from jax import lax
from jax.experimental import pallas as pl
from jax.experimental.pallas import tpu as pltpu
from jax.experimental.pallas import tpu_sc as plsc

_N = 4096
_C, _H, _W = 128, 7, 7
_NSLAB = _H * _W
_NC = 2
_NS = 16
_NW = _NC * _NS
_RPW = _N // _NW
_GROUPS = _RPW // 16
_NCB = 256


def _build():
    mesh = plsc.VectorSubcoreMesh(core_axis_name="c", subcore_axis_name="s")

    @functools.partial(
        pl.kernel,
        mesh=mesh,
        out_type=jax.ShapeDtypeStruct((_NSLAB, _N, _C), jnp.float32),
        scratch_types=[
            pltpu.VMEM((_N,), jnp.int32),
            pltpu.VMEM((16 * _N,), jnp.int32),
            pltpu.VMEM((16 * _NCB,), jnp.int32),
            pltpu.VMEM((_NCB,), jnp.int32),
            pltpu.VMEM((1, _RPW), jnp.int32),
            pltpu.VMEM((2, _RPW, _C), jnp.float32),
            pltpu.SemaphoreType.DMA,
            pltpu.SemaphoreType.DMA,
        ],
        compiler_params=pltpu.CompilerParams(needs_layout_passes=False),
    )
    def permute(x_hbm, tgt_hbm, out_hbm, key_v, fine_v, crs_v, crsp_v,
                idx_v, buf_v, sem_in, sem_out):
        wid = lax.axis_index("s") * _NC + lax.axis_index("c")
        base = wid * _RPW

        pltpu.sync_copy(tgt_hbm, key_v)

        for s0 in range(2):
            pltpu.async_copy(
                x_hbm.at[pl.ds(s0 * _N + base, _RPW)], buf_v.at[s0], sem_in)

        iota = lax.iota(jnp.int32, 16)
        z16 = iota * 0
        ones = z16 + 1
        loff_f = iota * _N
        loff_c = iota * _NCB

        def zf(k, _):
            for u in range(16):
                fine_v[pl.ds(k * 256 + u * 16, 16)] = z16
            return 0
        lax.fori_loop(0, 256, zf, 0)

        def zc(k, _):
            for u in range(4):
                crs_v[pl.ds(k * 64 + u * 16, 16)] = z16
            return 0
        lax.fori_loop(0, 64, zc, 0)

        def pa(jv, _):
            tv = key_v[pl.ds(jv * 16, 16)]
            m = (jv * 16 + iota) < base
            plsc.addupdate_scatter(fine_v, [loff_f + tv], ones, mask=m)
            plsc.addupdate_scatter(
                crs_v, [loff_c + lax.shift_right_logical(tv, 4)], ones)
            return 0
        lax.fori_loop(0, _N // 16, pa, 0)

        tg = [key_v[pl.ds(base + g * 16, 16)] for g in range(_GROUPS)]

        s1 = []
        for g in range(_GROUPS):
            acc = z16
            for lane in range(16):
                acc = acc + plsc.load_gather(fine_v, [tg[g] + lane * _N])
            s1.append(acc)

        def pb(jv, _):
            tv = key_v[pl.ds(jv * 16, 16)]
            m = (jv * 16 + iota) >= base
            plsc.addupdate_scatter(fine_v, [loff_f + tv], ones, mask=m)
            return 0
        lax.fori_loop(0, _N // 16, pb, 0)

        def cp(k, carry):
            m = z16
            for lane in range(16):
                m = m + crs_v[pl.ds(lane * _NCB + k * 16, 16)]
            incl = plsc.cumsum(m)
            crsp_v[pl.ds(k * 16, 16)] = incl - m + carry
            return carry + incl[15]
        lax.fori_loop(0, _NCB // 16, cp, jnp.int32(0))

        kc = [tg[g] * _RPW + (g * 16 + iota) for g in range(_GROUPS)]

        def tie(jb, accs):
            accA, accB = list(accs[0]), list(accs[1])
            kcb = key_v[pl.ds(base + jb * 16, 16)]
            kcc = kcb * _RPW + (jb * 16 + iota)
            for lane in range(16):
                kjc = kcc[lane]
                tj = kcb[lane]
                for g in range(_GROUPS):
                    accA[g] = accA[g] + lax.shift_right_logical(
                        kjc - kc[g], 31)
                    accB[g] = accB[g] + lax.shift_right_logical(
                        tj - tg[g], 31)
            return (tuple(accA), tuple(accB))
        accA, accB = lax.fori_loop(
            0, _GROUPS, tie,
            (tuple([z16] * _GROUPS), tuple([z16] * _GROUPS)))

        for g in range(_GROUPS):
            t_ = tg[g]
            r = jnp.bitwise_and(t_, 15)
            cb = lax.shift_right_logical(t_, 4)
            cb16 = t_ - r
            rank = (plsc.load_gather(crsp_v, [cb]) + s1[g]
                    + accA[g] - accB[g])

            def fine_term(o, rank):
                gs = z16
                for lane in range(16):
                    gs = gs + plsc.load_gather(
                        fine_v, [cb16 + (lane * _N + o)])
                return rank + gs * lax.shift_right_logical(o - r, 31)
            rank = lax.fori_loop(0, 16, fine_term, rank)

            plsc.store_scatter(idx_v, [z16, g * 16 + iota], rank)
        ranks = idx_v.at[0]

        def wait_in(b):
            pltpu.make_async_copy(
                x_hbm.at[pl.ds(0, _RPW)], buf_v.at[b], sem_in).wait()

        def wait_out(b):
            pltpu.make_async_copy(
                x_hbm.at[pl.ds(0, _RPW)], buf_v.at[b], sem_out).wait()

        wait_in(0)
        pltpu.async_copy(buf_v.at[0], out_hbm.at[0].at[ranks], sem_out)

        def pbody(s, _):
            b = s % 2
            wait_in(b)
            pltpu.async_copy(buf_v.at[b], out_hbm.at[s].at[ranks], sem_out)
            wait_out(1 - b)
            pltpu.async_copy(
                x_hbm.at[pl.ds((s + 1) * _N + base, _RPW)],
                buf_v.at[1 - b], sem_in)
            return 0
        lax.fori_loop(1, _NSLAB - 1, pbody, 0)

        s = _NSLAB - 1
        b = s % 2
        wait_in(b)
        pltpu.async_copy(buf_v.at[b], out_hbm.at[s].at[ranks], sem_out)
        wait_out(1 - b)
        wait_out(b)

    return permute


_permute = _build()


def kernel(rois, target):
    n, c, h, w = rois.shape
    x = rois.transpose(2, 3, 0, 1).reshape(h * w * n, c)
    out = _permute(x, target)
    return out.reshape(h, w, n, c).transpose(2, 3, 0, 1)


# speedup vs baseline: 2.2127x; 1.2343x over previous
import functools

import jax
import jax.numpy as jnp
from jax import lax
from jax.experimental import pallas as pl
from jax.experimental.pallas import tpu as pltpu
from jax.experimental.pallas import tpu_sc as plsc

_N = 4096
_C, _H, _W = 128, 7, 7
_NSLAB = _H * _W
_NC = 2
_NS = 16
_NW = _NC * _NS
_RPW = _N // _NW
_GROUPS = _RPW // 16
_NCB = 256


def _build():
    mesh = plsc.VectorSubcoreMesh(core_axis_name="c", subcore_axis_name="s")

    @functools.partial(
        pl.kernel,
        mesh=mesh,
        out_type=jax.ShapeDtypeStruct((_NSLAB, _N, _C), jnp.float32),
        scratch_types=[
            pltpu.VMEM((_N,), jnp.int32),
            pltpu.VMEM((8 * _N,), jnp.int32),
            pltpu.VMEM((16 * _NCB,), jnp.int32),
            pltpu.VMEM((_NCB,), jnp.int32),
            pltpu.VMEM((1, _RPW), jnp.int32),
            pltpu.VMEM((4, _RPW, _C), jnp.float32),
            pltpu.SemaphoreType.DMA,
            pltpu.SemaphoreType.DMA,
        ],
        compiler_params=pltpu.CompilerParams(needs_layout_passes=False),
    )
    def permute(x_hbm, tgt_hbm, out_hbm, key_v, fine_v, crs_v, crsp_v,
                idx_v, buf_v, sem_in, sem_out):
        wid = lax.axis_index("s") * _NC + lax.axis_index("c")
        base = wid * _RPW

        pltpu.sync_copy(tgt_hbm, key_v)

        for s0 in range(3):
            pltpu.async_copy(
                x_hbm.at[pl.ds(s0 * _N + base, _RPW)], buf_v.at[s0], sem_in)

        iota = lax.iota(jnp.int32, 16)
        z16 = iota * 0
        ones = z16 + 1
        loff_f = jnp.bitwise_and(iota, 7) * _N
        lo8 = iota < 8
        hi8 = iota >= 8
        loff_c = iota * _NCB

        def zf(k, _):
            for u in range(16):
                fine_v[pl.ds(k * 256 + u * 16, 16)] = z16
            return 0
        lax.fori_loop(0, 128, zf, 0)

        def zc(k, _):
            for u in range(4):
                crs_v[pl.ds(k * 64 + u * 16, 16)] = z16
            return 0
        lax.fori_loop(0, 64, zc, 0)

        def pa(jv, _):
            tv = key_v[pl.ds(jv * 16, 16)]
            m = (jv * 16 + iota) < base
            plsc.addupdate_scatter(fine_v, [loff_f + tv], ones,
                                   mask=jnp.logical_and(m, lo8))
            plsc.addupdate_scatter(fine_v, [loff_f + tv], ones,
                                   mask=jnp.logical_and(m, hi8))
            plsc.addupdate_scatter(
                crs_v, [loff_c + lax.shift_right_logical(tv, 4)], ones)
            return 0
        lax.fori_loop(0, _N // 16, pa, 0)

        tg = [key_v[pl.ds(base + g * 16, 16)] for g in range(_GROUPS)]

        s1 = []
        for g in range(_GROUPS):
            acc = z16
            for lane in range(8):
                acc = acc + plsc.load_gather(fine_v, [tg[g] + lane * _N])
            s1.append(acc)

        def pb(jv, _):
            tv = key_v[pl.ds(jv * 16, 16)]
            m = (jv * 16 + iota) >= base
            plsc.addupdate_scatter(fine_v, [loff_f + tv], ones,
                                   mask=jnp.logical_and(m, lo8))
            plsc.addupdate_scatter(fine_v, [loff_f + tv], ones,
                                   mask=jnp.logical_and(m, hi8))
            return 0
        lax.fori_loop(0, _N // 16, pb, 0)

        def cp(k, carry):
            m = z16
            for lane in range(16):
                m = m + crs_v[pl.ds(lane * _NCB + k * 16, 16)]
            incl = plsc.cumsum(m)
            crsp_v[pl.ds(k * 16, 16)] = incl - m + carry
            return carry + incl[15]
        lax.fori_loop(0, _NCB // 16, cp, jnp.int32(0))

        kc = [tg[g] * _RPW + (g * 16 + iota) for g in range(_GROUPS)]

        def tie(jb, accs):
            accA, accB = list(accs[0]), list(accs[1])
            kcb = key_v[pl.ds(base + jb * 16, 16)]
            kcc = kcb * _RPW + (jb * 16 + iota)
            for lane in range(16):
                kjc = kcc[lane]
                tj = kcb[lane]
                for g in range(_GROUPS):
                    accA[g] = accA[g] + lax.shift_right_logical(
                        kjc - kc[g], 31)
                    accB[g] = accB[g] + lax.shift_right_logical(
                        tj - tg[g], 31)
            return (tuple(accA), tuple(accB))
        accA, accB = lax.fori_loop(
            0, _GROUPS, tie,
            (tuple([z16] * _GROUPS), tuple([z16] * _GROUPS)))

        for g in range(_GROUPS):
            t_ = tg[g]
            r = jnp.bitwise_and(t_, 15)
            cb = lax.shift_right_logical(t_, 4)
            cb16 = t_ - r
            rank = (plsc.load_gather(crsp_v, [cb]) + s1[g]
                    + accA[g] - accB[g])

            def fine_term(o, rank):
                gs = z16
                for lane in range(8):
                    gs = gs + plsc.load_gather(
                        fine_v, [cb16 + (lane * _N + o)])
                return rank + gs * lax.shift_right_logical(o - r, 31)
            rank = lax.fori_loop(0, 16, fine_term, rank)

            plsc.store_scatter(idx_v, [z16, g * 16 + iota], rank)
        ranks = idx_v.at[0]

        def wait_in(b):
            pltpu.make_async_copy(
                x_hbm.at[pl.ds(0, _RPW)], buf_v.at[b], sem_in).wait()

        def wait_out(b):
            pltpu.make_async_copy(
                x_hbm.at[pl.ds(0, _RPW)], buf_v.at[b], sem_out).wait()

        wait_in(0)
        pltpu.async_copy(buf_v.at[0], out_hbm.at[0].at[ranks], sem_out)
        pltpu.async_copy(
            x_hbm.at[pl.ds(3 * _N + base, _RPW)], buf_v.at[3], sem_in)

        def pbody(s, _):
            b = s % 4
            wait_in(b)
            pltpu.async_copy(buf_v.at[b], out_hbm.at[s].at[ranks], sem_out)
            wait_out((s - 1) % 4)
            pltpu.async_copy(
                x_hbm.at[pl.ds((s + 3) * _N + base, _RPW)],
                buf_v.at[(s + 3) % 4], sem_in)
            return 0
        lax.fori_loop(1, _NSLAB - 3, pbody, 0)

        for s in range(_NSLAB - 3, _NSLAB):
            b = s % 4
            wait_in(b)
            pltpu.async_copy(buf_v.at[b], out_hbm.at[s].at[ranks], sem_out)
            wait_out((s - 1) % 4)
        wait_out((_NSLAB - 1) % 4)

    return permute


_permute = _build()


def kernel(rois, target):
    n, c, h, w = rois.shape
    x = rois.transpose(2, 3, 0, 1).reshape(h * w * n, c)
    out = _permute(x, target)
    return out.reshape(h, w, n, c).transpose(2, 3, 0, 1)
